# baseline (speedup 1.0000x reference)
import functools

import jax
import jax.numpy as jnp
from jax import lax
from jax.experimental import pallas as pl
from jax.experimental.pallas import tpu as pltpu
from jax.experimental.pallas import tpu_sc as plsc

L = 16
SC_CHUNKS_PER_WORKER = 8
R = 128
TC_BLOCK = 512


def _sc_shard(X2, idx, n_sc, NC, NS):
    N, C = X2.shape
    K, G = idx.shape
    NW = NC * NS
    rows_per_w = n_sc // NW
    nchunk = rows_per_w // R
    ngroups = K // L
    scale = 1.0 / G

    mesh = plsc.VectorSubcoreMesh(core_axis_name="c", subcore_axis_name="s")

    @functools.partial(
        pl.kernel,
        mesh=mesh,
        compiler_params=pltpu.CompilerParams(needs_layout_passes=False),
        out_type=jax.ShapeDtypeStruct((n_sc, K), jnp.float32),
        scratch_types=[
            pltpu.VMEM((R, C), jnp.float32),
            pltpu.VMEM((R, C), jnp.float32),
            pltpu.VMEM((R, K), jnp.float32),
            pltpu.VMEM((R, K), jnp.float32),
            pltpu.VMEM((K, G), jnp.int32),
            pltpu.SemaphoreType.DMA,
            pltpu.SemaphoreType.DMA,
            pltpu.SemaphoreType.DMA,
            pltpu.SemaphoreType.DMA,
        ],
    )
    def _unpool(x_hbm, idx_hbm, out_hbm, x_v0, x_v1, o_v0, o_v1, idx_v,
                si0, si1, so0, so1):
        xb, ob, si, so = [x_v0, x_v1], [o_v0, o_v1], [si0, si1], [so0, so1]
        wid = lax.axis_index("s") * NC + lax.axis_index("c")
        base = wid * rows_per_w
        pltpu.sync_copy(idx_hbm, idx_v)
        lanes = lax.iota(jnp.int32, L)
        cols = [[plsc.load_gather(idx_v,
                                  [lanes + g * L,
                                   jnp.full((L,), j, dtype=jnp.int32)])
                 for g in range(ngroups)] for j in range(G)]

        def in_copy(c, b):
            return pltpu.make_async_copy(
                x_hbm.at[pl.ds(base + c * R, R)], xb[b], si[b])

        def out_copy(c, b):
            return pltpu.make_async_copy(
                ob[b], out_hbm.at[pl.ds(base + c * R, R)], so[b])

        def compute(x_v, o_v):
            @plsc.parallel_loop(0, R, 1)
            def row_body(r):
                rvec = jnp.full((L,), r, dtype=jnp.int32)
                gath = [[plsc.load_gather(x_v, [rvec, cols[j][g]])
                         for g in range(ngroups)] for j in range(G)]
                for g in range(ngroups):
                    acc = gath[0][g]
                    for j in range(1, G):
                        acc = acc + gath[j][g]
                    o_v[r, pl.ds(g * L, L)] = acc * scale

        in_copy(0, 0).start()
        in_copy(1, 1).start()

        def super_body(sc, carry):
            for b in range(2):
                c = sc * 2 + b
                in_copy(c, b).wait()

                @pl.when(sc > 0)
                def _():
                    out_copy(c - 2, b).wait()

                compute(xb[b], ob[b])
                out_copy(c, b).start()

                @pl.when(c + 2 < nchunk)
                def _():
                    in_copy(c + 2, b).start()

            return carry

        lax.fori_loop(0, nchunk // 2, super_body, 0)
        out_copy(nchunk - 2, 0).wait()
        out_copy(nchunk - 1, 1).wait()

    return _unpool(X2, idx)


def _tc_shard(X2, idx, n_sc):
    N, C = X2.shape
    K, G = idx.shape
    n_tc = N - n_sc
    scale = 1.0 / G
    blk0 = n_sc // TC_BLOCK

    def body(idx_ref, x_ref, o_ref):
        c_iota = lax.broadcasted_iota(jnp.int32, (C, K), 0)
        m = jnp.zeros((C, K), jnp.float32)
        for j in range(G):
            colj = idx_ref[:, j].reshape(1, K)
            m = m + jnp.where(c_iota == colj, 1.0, 0.0)
        o_ref[...] = jnp.dot(x_ref[...], m * scale,
                             preferred_element_type=jnp.float32)

    return pl.pallas_call(
        body,
        grid=(n_tc // TC_BLOCK,),
        in_specs=[
            pl.BlockSpec((K, G), lambda i: (0, 0)),
            pl.BlockSpec((TC_BLOCK, C), lambda i: (blk0 + i, 0)),
        ],
        out_specs=pl.BlockSpec((TC_BLOCK, K), lambda i: (i, 0)),
        out_shape=jax.ShapeDtypeStruct((n_tc, K), jnp.float32),
    )(idx, X2)


def kernel(x, idx):
    B, Ch, C = x.shape
    K, G = idx.shape
    N = B * Ch
    X2 = x.reshape(N, C)

    info = plsc.get_sparse_core_info()
    NC, NS = info.num_cores, info.num_subcores
    n_sc = NC * NS * SC_CHUNKS_PER_WORKER * R

    sc_out = _sc_shard(X2, idx, n_sc, NC, NS)
    tc_out = _tc_shard(X2, idx, n_sc)
    out = jnp.concatenate([sc_out, tc_out], axis=0)
    return out.reshape(B, Ch, K)


# speedup vs baseline: 1.2892x; 1.2892x over previous
import functools

import jax
import jax.numpy as jnp
from jax import lax
from jax.experimental import pallas as pl
from jax.experimental.pallas import tpu as pltpu
from jax.experimental.pallas import tpu_sc as plsc

L = 16
SC_CHUNKS_PER_WORKER = 8
R = 128
TC_BLOCK = 2048


def _sc_shard(X2, idx, n_sc, NC, NS):
    N, C = X2.shape
    K, G = idx.shape
    NW = NC * NS
    rows_per_w = n_sc // NW
    nchunk = rows_per_w // R
    ngroups = K // L
    scale = 1.0 / G

    mesh = plsc.VectorSubcoreMesh(core_axis_name="c", subcore_axis_name="s")

    @functools.partial(
        pl.kernel,
        mesh=mesh,
        compiler_params=pltpu.CompilerParams(needs_layout_passes=False),
        out_type=jax.ShapeDtypeStruct((n_sc, K), jnp.float32),
        scratch_types=[
            pltpu.VMEM((R, C), jnp.float32),
            pltpu.VMEM((R, C), jnp.float32),
            pltpu.VMEM((R, K), jnp.float32),
            pltpu.VMEM((R, K), jnp.float32),
            pltpu.VMEM((K, G), jnp.int32),
            pltpu.SemaphoreType.DMA,
            pltpu.SemaphoreType.DMA,
            pltpu.SemaphoreType.DMA,
            pltpu.SemaphoreType.DMA,
        ],
    )
    def _unpool(x_hbm, idx_hbm, out_hbm, x_v0, x_v1, o_v0, o_v1, idx_v,
                si0, si1, so0, so1):
        xb, ob, si, so = [x_v0, x_v1], [o_v0, o_v1], [si0, si1], [so0, so1]
        wid = lax.axis_index("s") * NC + lax.axis_index("c")
        base = wid * rows_per_w
        pltpu.sync_copy(idx_hbm, idx_v)
        lanes = lax.iota(jnp.int32, L)
        cols = [[plsc.load_gather(idx_v,
                                  [lanes + g * L,
                                   jnp.full((L,), j, dtype=jnp.int32)])
                 for g in range(ngroups)] for j in range(G)]

        def in_copy(c, b):
            return pltpu.make_async_copy(
                x_hbm.at[pl.ds(base + c * R, R)], xb[b], si[b])

        def out_copy(c, b):
            return pltpu.make_async_copy(
                ob[b], out_hbm.at[pl.ds(base + c * R, R)], so[b])

        def compute(x_v, o_v):
            @plsc.parallel_loop(0, R, 1)
            def row_body(r):
                rvec = jnp.full((L,), r, dtype=jnp.int32)
                gath = [[plsc.load_gather(x_v, [rvec, cols[j][g]])
                         for g in range(ngroups)] for j in range(G)]
                for g in range(ngroups):
                    acc = gath[0][g]
                    for j in range(1, G):
                        acc = acc + gath[j][g]
                    o_v[r, pl.ds(g * L, L)] = acc * scale

        in_copy(0, 0).start()
        in_copy(1, 1).start()

        def super_body(sc, carry):
            for b in range(2):
                c = sc * 2 + b
                in_copy(c, b).wait()

                @pl.when(sc > 0)
                def _():
                    out_copy(c - 2, b).wait()

                compute(xb[b], ob[b])
                out_copy(c, b).start()

                @pl.when(c + 2 < nchunk)
                def _():
                    in_copy(c + 2, b).start()

            return carry

        lax.fori_loop(0, nchunk // 2, super_body, 0)
        out_copy(nchunk - 2, 0).wait()
        out_copy(nchunk - 1, 1).wait()

    return _unpool(X2, idx)


def _tc_shard(X2, idx, n_sc):
    N, C = X2.shape
    K, G = idx.shape
    n_tc = N - n_sc
    scale = 1.0 / G
    blk0 = n_sc // TC_BLOCK

    def body(idx_ref, x_ref, o_ref):
        c_iota = lax.broadcasted_iota(jnp.int32, (C, K), 0)
        m = jnp.zeros((C, K), jnp.float32)
        for j in range(G):
            colj = idx_ref[:, j].reshape(1, K)
            m = m + jnp.where(c_iota == colj, 1.0, 0.0)
        o_ref[...] = jnp.dot(x_ref[...], m * scale,
                             preferred_element_type=jnp.float32)

    return pl.pallas_call(
        body,
        grid=(n_tc // TC_BLOCK,),
        in_specs=[
            pl.BlockSpec((K, G), lambda i: (0, 0)),
            pl.BlockSpec((TC_BLOCK, C), lambda i: (blk0 + i, 0)),
        ],
        out_specs=pl.BlockSpec((TC_BLOCK, K), lambda i: (i, 0)),
        out_shape=jax.ShapeDtypeStruct((n_tc, K), jnp.float32),
    )(idx, X2)


def kernel(x, idx):
    B, Ch, C = x.shape
    K, G = idx.shape
    N = B * Ch
    X2 = x.reshape(N, C)

    info = plsc.get_sparse_core_info()
    NC, NS = info.num_cores, info.num_subcores
    n_sc = NC * NS * SC_CHUNKS_PER_WORKER * R

    sc_out = _sc_shard(X2, idx, n_sc, NC, NS)
    tc_out = _tc_shard(X2, idx, n_sc)
    out = jnp.concatenate([sc_out, tc_out], axis=0)
    return out.reshape(B, Ch, K)


# speedup vs baseline: 1.6449x; 1.2760x over previous
import functools

import jax
import jax.numpy as jnp
from jax import lax
from jax.experimental import pallas as pl
from jax.experimental.pallas import tpu as pltpu
from jax.experimental.pallas import tpu_sc as plsc

L = 16


def kernel(x, idx):
    B, Ch, C = x.shape
    K, G = idx.shape
    N = B * Ch
    X2 = x.reshape(N, C)

    info = plsc.get_sparse_core_info()
    NC, NS = info.num_cores, info.num_subcores
    NW = NC * NS
    rows_per_w = N // NW
    R = 128
    nchunk = rows_per_w // R
    ngroups = K // L
    scale = 1.0 / G

    mesh = plsc.VectorSubcoreMesh(core_axis_name="c", subcore_axis_name="s")

    @functools.partial(
        pl.kernel,
        mesh=mesh,
        compiler_params=pltpu.CompilerParams(needs_layout_passes=False),
        out_type=jax.ShapeDtypeStruct((N, K), jnp.float32),
        scratch_types=[
            pltpu.VMEM((R, C), jnp.float32),
            pltpu.VMEM((R, C), jnp.float32),
            pltpu.VMEM((R, K), jnp.float32),
            pltpu.VMEM((R, K), jnp.float32),
            pltpu.VMEM((K, G), jnp.int32),
            pltpu.SemaphoreType.DMA,
            pltpu.SemaphoreType.DMA,
            pltpu.SemaphoreType.DMA,
            pltpu.SemaphoreType.DMA,
        ],
    )
    def _unpool(x_hbm, idx_hbm, out_hbm, x_v0, x_v1, o_v0, o_v1, idx_v,
                si0, si1, so0, so1):
        xb, ob, si, so = [x_v0, x_v1], [o_v0, o_v1], [si0, si1], [so0, so1]
        wid = lax.axis_index("s") * NC + lax.axis_index("c")
        base = wid * rows_per_w
        pltpu.sync_copy(idx_hbm, idx_v)
        lanes = lax.iota(jnp.int32, L)
        cols = [[plsc.load_gather(idx_v,
                                  [lanes + g * L,
                                   jnp.full((L,), j, dtype=jnp.int32)])
                 for g in range(ngroups)] for j in range(G)]

        def in_copy(c, b):
            return pltpu.make_async_copy(
                x_hbm.at[pl.ds(base + c * R, R)], xb[b], si[b])

        def out_copy(c, b):
            return pltpu.make_async_copy(
                ob[b], out_hbm.at[pl.ds(base + c * R, R)], so[b])

        def compute(x_v, o_v):
            @plsc.parallel_loop(0, R, 1)
            def row_body(r):
                rvec = jnp.full((L,), r, dtype=jnp.int32)
                gath = [[plsc.load_gather(x_v, [rvec, cols[j][g]])
                         for g in range(ngroups)] for j in range(G)]
                for g in range(ngroups):
                    acc = gath[0][g]
                    for j in range(1, G):
                        acc = acc + gath[j][g]
                    o_v[r, pl.ds(g * L, L)] = acc * scale

        in_copy(0, 0).start()
        in_copy(1, 1).start()

        def super_body(sc, carry):
            for b in range(2):
                c = sc * 2 + b
                in_copy(c, b).wait()

                @pl.when(sc > 0)
                def _():
                    out_copy(c - 2, b).wait()

                compute(xb[b], ob[b])
                out_copy(c, b).start()

                @pl.when(c + 2 < nchunk)
                def _():
                    in_copy(c + 2, b).start()

            return carry

        lax.fori_loop(0, nchunk // 2, super_body, 0)
        out_copy(nchunk - 2, 0).wait()
        out_copy(nchunk - 1, 1).wait()

    out = _unpool(X2, idx)
    return out.reshape(B, Ch, K)
